# baseline (speedup 1.0000x reference)
import functools

import jax
import jax.numpy as jnp
from jax import lax
from jax.experimental import pallas as pl
from jax.experimental.pallas import tpu as pltpu
from jax.experimental.pallas import tpu_sc as plsc

B = 16384
D = 64
L = 16
NC, NS = 2, 16
NW = NC * NS
BPW = B // NW
GCH = 128
NCHUNK = BPW // GCH
NROWS = 1000000
TBLK = 16384
TSTEPS = -(-NROWS // TBLK)


def _tc_scan_body(u_ref, i_ref, w2_ref, su_ref, si_ref):
    w2 = w2_ref[...]
    ru = lax.dot_general(w2, u_ref[...], (((0,), (1,)), ((), ())),
                         preferred_element_type=jnp.float32)
    ri = lax.dot_general(w2, i_ref[...], (((0,), (1,)), ((), ())),
                         preferred_element_type=jnp.float32)
    su_ref[...] = ru[0]
    si_ref[...] = ri[1]


@jax.jit
def _tc_scan(u_emb, i_emb, w2):
    return pl.pallas_call(
        _tc_scan_body,
        grid=(TSTEPS,),
        in_specs=[
            pl.BlockSpec((TBLK, D), lambda i: (i, 0)),
            pl.BlockSpec((TBLK, D), lambda i: (i, 0)),
            pl.BlockSpec((D, 2), lambda i: (0, 0)),
        ],
        out_specs=[
            pl.BlockSpec((TBLK,), lambda i: (i,)),
            pl.BlockSpec((TBLK,), lambda i: (i,)),
        ],
        out_shape=[
            jax.ShapeDtypeStruct((NROWS,), jnp.float32),
            jax.ShapeDtypeStruct((NROWS,), jnp.float32),
        ],
    )(u_emb, i_emb, w2)


def _sc_gather_impl(su_hbm, si_hbm, uid_hbm, iid_hbm, bias_hbm, out_hbm,
                    uidx_v, iidx_v, ubuf, ibuf, outv, bv, sem):
    wid = lax.axis_index("s") * NC + lax.axis_index("c")
    base = wid * BPW

    pltpu.sync_copy(uid_hbm.at[wid], uidx_v)
    pltpu.sync_copy(iid_hbm.at[wid], iidx_v)
    pltpu.sync_copy(bias_hbm, bv)

    copies = []
    for j in range(NCHUNK):
        copies.append(pltpu.async_copy(
            su_hbm.at[uidx_v.at[j]], ubuf.at[pl.ds(j * GCH, GCH)], sem))
        copies.append(pltpu.async_copy(
            si_hbm.at[iidx_v.at[j]], ibuf.at[pl.ds(j * GCH, GCH)], sem))
    for c in copies:
        c.wait()

    bias = bv[...]
    for k in range(BPW // L):
        outv[pl.ds(k * L, L)] = (
            ubuf[pl.ds(k * L, L)] + ibuf[pl.ds(k * L, L)] + bias)

    pltpu.sync_copy(outv, out_hbm.at[pl.ds(base, BPW)])


@functools.cache
def _build_sc_gather():
    mesh = plsc.VectorSubcoreMesh(
        core_axis_name="c", subcore_axis_name="s",
        num_cores=NC, num_subcores=NS,
    )
    return pl.kernel(
        _sc_gather_impl,
        out_type=jax.ShapeDtypeStruct((B,), jnp.float32),
        mesh=mesh,
        scratch_types=[
            pltpu.VMEM((NCHUNK, GCH), jnp.int32),
            pltpu.VMEM((NCHUNK, GCH), jnp.int32),
            pltpu.VMEM((BPW,), jnp.float32),
            pltpu.VMEM((BPW,), jnp.float32),
            pltpu.VMEM((BPW,), jnp.float32),
            pltpu.VMEM((L,), jnp.float32),
            pltpu.SemaphoreType.DMA,
        ],
        compiler_params=pltpu.CompilerParams(use_tc_tiling_on_sc=False),
    )


def kernel(train_x, u_emb, i_emb, W, b):
    uid = train_x[:, 0].reshape(NW, NCHUNK, GCH)
    iid = train_x[:, 1].reshape(NW, NCHUNK, GCH)
    w2 = W.reshape(2, D).T.astype(jnp.float32)
    bias16 = jnp.full((L,), b.reshape(-1)[0], jnp.float32)
    su, si = _tc_scan(u_emb, i_emb, w2)
    return _build_sc_gather()(su, si, uid, iid, bias16)


# speedup vs baseline: 1.3436x; 1.3436x over previous
import functools

import jax
import jax.numpy as jnp
from jax import lax
from jax.experimental import pallas as pl
from jax.experimental.pallas import tpu as pltpu
from jax.experimental.pallas import tpu_sc as plsc

B = 16384
D = 64
L = 16
NC, NS = 2, 16
NW = NC * NS
BPW = B // NW
NG = BPW // L

_BREV = [int(f"{s:04b}"[::-1], 2) for s in range(L)]


def _sc_fwd_impl(uid_hbm, iid_hbm, uemb_hbm, iemb_hbm, wext_hbm, dummy_hbm,
                 out_hbm, uidx_v, iidx_v, tbu0, tbu1, tbi0, tbi1, wv, outv,
                 sem):
    wid = lax.axis_index("s") * NC + lax.axis_index("c")
    base = wid * BPW

    pltpu.sync_copy(uid_hbm.at[wid], uidx_v)
    pltpu.sync_copy(iid_hbm.at[wid], iidx_v)
    pltpu.sync_copy(wext_hbm, wv)

    def rvecs(g):
        return uidx_v[pl.ds(g * L, L)], iidx_v[pl.ds(g * L, L)]

    def fire(g, tbu, tbi):
        uvec, ivec = rvecs(g)
        for s in range(L):
            ur = uvec[s]
            ir = ivec[s]
            ut = pl.multiple_of((ur >> 3) * 8, 8)
            it = pl.multiple_of((ir >> 3) * 8, 8)
            pltpu.make_async_copy(
                uemb_hbm.at[pl.ds(ut, 8), :], tbu.at[s], sem).start()
            pltpu.make_async_copy(
                iemb_hbm.at[pl.ds(it, 8), :], tbi.at[s], sem).start()

    def wait_group(tbu, tbi):
        pltpu.make_async_copy(dummy_hbm, tbu, sem).wait()
        pltpu.make_async_copy(dummy_hbm, tbi, sem).wait()

    wchunks = [wv[pl.ds(16 * j, L)] for j in range(8)]
    bias = wv[pl.ds(128, L)]
    iota = lax.iota(jnp.int32, L)

    def dg(v, idx):
        return lax.gather(
            v, idx.reshape(L, 1),
            dimension_numbers=lax.GatherDimensionNumbers(
                offset_dims=(), collapsed_slice_dims=(0,), start_index_map=(0,)),
            slice_sizes=(1,),
            mode=lax.GatherScatterMode.PROMISE_IN_BOUNDS,
        )

    def compute(g, tbu, tbi):
        uvec, ivec = rvecs(g)
        vecs = []
        for s in range(L):
            slot = _BREV[s]
            rru = uvec[slot] & 7
            rri = ivec[slot] & 7
            t = tbu[slot, rru, pl.ds(0, L)] * wchunks[0]
            for j in range(1, 4):
                t = t + tbu[slot, rru, pl.ds(16 * j, L)] * wchunks[j]
            for j in range(4):
                t = t + tbi[slot, rri, pl.ds(16 * j, L)] * wchunks[4 + j]
            vecs.append(t)
        for h in (8, 4, 2, 1):
            folded = [v + dg(v, iota ^ h) for v in vecs]
            vecs = [
                jnp.where((iota & h) == 0, folded[2 * p], folded[2 * p + 1])
                for p in range(len(folded) // 2)
            ]
        outv[pl.ds(g * L, L)] = vecs[0] + bias

    fire(0, tbu0, tbi0)
    fire(1, tbu1, tbi1)

    def body(gg, _):
        g0 = gg * 2
        wait_group(tbu0, tbi0)
        compute(g0, tbu0, tbi0)

        @pl.when(g0 + 2 < NG)
        def _f0():
            fire(g0 + 2, tbu0, tbi0)

        wait_group(tbu1, tbi1)
        compute(g0 + 1, tbu1, tbi1)

        @pl.when(g0 + 3 < NG)
        def _f1():
            fire(g0 + 3, tbu1, tbi1)

        return _

    lax.fori_loop(0, NG // 2, body, 0)
    pltpu.sync_copy(outv, out_hbm.at[pl.ds(base, BPW)])


@functools.cache
def _build_sc_fwd():
    mesh = plsc.VectorSubcoreMesh(
        core_axis_name="c", subcore_axis_name="s",
        num_cores=NC, num_subcores=NS,
    )
    return pl.kernel(
        _sc_fwd_impl,
        out_type=jax.ShapeDtypeStruct((B,), jnp.float32),
        mesh=mesh,
        scratch_types=[
            pltpu.VMEM((BPW,), jnp.int32),
            pltpu.VMEM((BPW,), jnp.int32),
            pltpu.VMEM((L, 8, D), jnp.float32),
            pltpu.VMEM((L, 8, D), jnp.float32),
            pltpu.VMEM((L, 8, D), jnp.float32),
            pltpu.VMEM((L, 8, D), jnp.float32),
            pltpu.VMEM((144,), jnp.float32),
            pltpu.VMEM((BPW,), jnp.float32),
            pltpu.SemaphoreType.DMA,
        ],
    )


def kernel(train_x, u_emb, i_emb, W, b):
    uid = train_x[:, 0].reshape(NW, BPW)
    iid = train_x[:, 1].reshape(NW, BPW)
    wext = jnp.concatenate(
        [W.reshape(-1), jnp.broadcast_to(b.reshape(-1)[0], (L,))]
    ).astype(jnp.float32)
    dummy = jnp.zeros((L, 8, D), jnp.float32)
    return _build_sc_fwd()(uid, iid, u_emb, i_emb, wext, dummy)


# speedup vs baseline: 5.6049x; 4.1715x over previous
import functools

import jax
import jax.numpy as jnp
from jax import lax
from jax.experimental import pallas as pl
from jax.experimental.pallas import tpu as pltpu
from jax.experimental.pallas import tpu_sc as plsc

B = 16384
D = 64
L = 16
NC, NS = 2, 16
NW = NC * NS
BPW = B // NW
GCH = 128
NCHUNK = BPW // GCH
NROWS = 1000000
TBLKC = 16384
TSTEPS = -(-NROWS // TBLKC)


def _tc_scan_body(u_ref, i_ref, w2_ref, su_ref, si_ref):
    w2 = w2_ref[...]
    ru = lax.dot_general(w2, u_ref[...], (((1,), (0,)), ((), ())),
                         preferred_element_type=jnp.float32)
    ri = lax.dot_general(w2, i_ref[...], (((1,), (0,)), ((), ())),
                         preferred_element_type=jnp.float32)
    su_ref[...] = ru[0]
    si_ref[...] = ri[1]


def _tc_scan(ut, it, w2):
    return pl.pallas_call(
        _tc_scan_body,
        grid=(TSTEPS,),
        in_specs=[
            pl.BlockSpec((D, TBLKC), lambda i: (0, i)),
            pl.BlockSpec((D, TBLKC), lambda i: (0, i)),
            pl.BlockSpec((2, D), lambda i: (0, 0)),
        ],
        out_specs=[
            pl.BlockSpec((TBLKC,), lambda i: (i,)),
            pl.BlockSpec((TBLKC,), lambda i: (i,)),
        ],
        out_shape=[
            jax.ShapeDtypeStruct((NROWS,), jnp.float32),
            jax.ShapeDtypeStruct((NROWS,), jnp.float32),
        ],
    )(ut, it, w2)


def _sc_gather_impl(su_hbm, si_hbm, uid_hbm, iid_hbm, bias_hbm, out_hbm,
                    uidx_v, iidx_v, ubuf, ibuf, outv, bv, sem):
    wid = lax.axis_index("s") * NC + lax.axis_index("c")
    base = wid * BPW

    pltpu.sync_copy(uid_hbm.at[wid], uidx_v)
    pltpu.sync_copy(iid_hbm.at[wid], iidx_v)
    pltpu.sync_copy(bias_hbm, bv)

    copies = []
    for j in range(NCHUNK):
        copies.append(pltpu.async_copy(
            su_hbm.at[uidx_v.at[j]], ubuf.at[pl.ds(j * GCH, GCH)], sem))
        copies.append(pltpu.async_copy(
            si_hbm.at[iidx_v.at[j]], ibuf.at[pl.ds(j * GCH, GCH)], sem))
    for c in copies:
        c.wait()

    bias = bv[...]
    for k in range(BPW // L):
        outv[pl.ds(k * L, L)] = (
            ubuf[pl.ds(k * L, L)] + ibuf[pl.ds(k * L, L)] + bias)

    pltpu.sync_copy(outv, out_hbm.at[pl.ds(base, BPW)])


@functools.cache
def _build_sc_gather():
    mesh = plsc.VectorSubcoreMesh(
        core_axis_name="c", subcore_axis_name="s",
        num_cores=NC, num_subcores=NS,
    )
    return pl.kernel(
        _sc_gather_impl,
        out_type=jax.ShapeDtypeStruct((B,), jnp.float32),
        mesh=mesh,
        scratch_types=[
            pltpu.VMEM((NCHUNK, GCH), jnp.int32),
            pltpu.VMEM((NCHUNK, GCH), jnp.int32),
            pltpu.VMEM((BPW,), jnp.float32),
            pltpu.VMEM((BPW,), jnp.float32),
            pltpu.VMEM((BPW,), jnp.float32),
            pltpu.VMEM((L,), jnp.float32),
            pltpu.SemaphoreType.DMA,
        ],
        compiler_params=pltpu.CompilerParams(use_tc_tiling_on_sc=False),
    )


def kernel(train_x, u_emb, i_emb, W, b):
    uid = train_x[:, 0].reshape(NW, NCHUNK, GCH)
    iid = train_x[:, 1].reshape(NW, NCHUNK, GCH)
    w2 = W.reshape(2, D).astype(jnp.float32)
    bias16 = jnp.full((L,), b.reshape(-1)[0], jnp.float32)
    su, si = _tc_scan(u_emb.T, i_emb.T, w2)
    return _build_sc_gather()(su, si, uid, iid, bias16)
